# initial kernel scaffold (unmeasured)
import functools

import jax
import jax.numpy as jnp
from jax import lax
from jax.experimental import pallas as pl
from jax.experimental.pallas import tpu as pltpu

N_DEV = 32
B, SQ, D, DH = 2, 256, 768, 64
HQ_LOCAL = 8
GQA = 4
KV_LOCAL = HQ_LOCAL // GQA
KVCOLS = KV_LOCAL * DH
TOK = B * SQ
CH = TOK // N_DEV
NH = N_DEV - 1


def kernel(x, Wq, Wo, Wk, Wv):
    me_out = lax.axis_index("i")
    xf = x.reshape(TOK, D)
    Wk_s = lax.dynamic_slice(Wk, (0, me_out * KVCOLS), (D, KVCOLS))
    Wv_s = lax.dynamic_slice(Wv, (0, me_out * KVCOLS), (D, KVCOLS))

    def body(x_ref, wq_ref, wk_ref, wv_ref, wo_ref, out_ref,
             part_ref, rbuf_ref, rs_send, rs_recv, ag_send, ag_recv):
        me = lax.axis_index("i")
        left = lax.rem(me + N_DEV - 1, N_DEV)
        right = lax.rem(me + 1, N_DEV)

        barrier = pltpu.get_barrier_semaphore()
        for nbr in (left, right):
            pl.semaphore_signal(barrier, inc=1, device_id=(nbr,),
                                device_id_type=pl.DeviceIdType.MESH)
        pl.semaphore_wait(barrier, 2)

        xv = x_ref[:, :]
        q = jnp.dot(xv, wq_ref[:, :], preferred_element_type=jnp.float32)
        kk = jnp.dot(xv, wk_ref[:, :], preferred_element_type=jnp.float32)
        vv = jnp.dot(xv, wv_ref[:, :], preferred_element_type=jnp.float32)
        o_rows = []
        for b in range(B):
            r0 = b * SQ
            heads = []
            for h in range(HQ_LOCAL):
                g = h // GQA
                qh = q[r0:r0 + SQ, h * DH:(h + 1) * DH]
                kh = kk[r0:r0 + SQ, g * DH:(g + 1) * DH]
                vh = vv[r0:r0 + SQ, g * DH:(g + 1) * DH]
                s = lax.dot_general(qh, kh, (((1,), (1,)), ((), ())),
                                    preferred_element_type=jnp.float32)
                s = s * 0.125
                m = jnp.max(s, axis=1, keepdims=True)
                p = jnp.exp(s - m)
                l = jnp.sum(p, axis=1, keepdims=True)
                o = jnp.dot(p, vh, preferred_element_type=jnp.float32) / l
                heads.append(o)
            o_rows.append(jnp.concatenate(heads, axis=1))
        o_full = jnp.concatenate(o_rows, axis=0)
        part_ref[:, :] = jnp.dot(o_full, wo_ref[:, :],
                                 preferred_element_type=jnp.float32)

        rs = []
        for t in range(NH):
            c = lax.rem(me - t + N_DEV, N_DEV)
            if t == 0:
                src = part_ref.at[pl.ds(c * CH, CH), :]
            else:
                rs[t - 1].wait_recv()
                rbuf_ref[t - 1, :, :] = (
                    rbuf_ref[t - 1, :, :] + part_ref[pl.ds(c * CH, CH), :])
                src = rbuf_ref.at[t - 1]
            d = pltpu.make_async_remote_copy(
                src_ref=src, dst_ref=rbuf_ref.at[t],
                send_sem=rs_send.at[t], recv_sem=rs_recv.at[t],
                device_id=(right,), device_id_type=pl.DeviceIdType.MESH)
            d.start()
            rs.append(d)
        rs[NH - 1].wait_recv()
        c_own = lax.rem(me + 1, N_DEV)
        out_ref[pl.ds(c_own * CH, CH), :] = (
            rbuf_ref[NH - 1, :, :] + part_ref[pl.ds(c_own * CH, CH), :])

        ag = []
        for t in range(NH):
            c_src = lax.rem(me + 1 - t + N_DEV, N_DEV)
            c_dst = lax.rem(me - t + N_DEV, N_DEV)
            if t >= 1:
                ag[t - 1].wait_recv()
            d = pltpu.make_async_remote_copy(
                src_ref=out_ref.at[pl.ds(c_src * CH, CH), :],
                dst_ref=out_ref.at[pl.ds(c_dst * CH, CH), :],
                send_sem=ag_send.at[t], recv_sem=ag_recv.at[t],
                device_id=(right,), device_id_type=pl.DeviceIdType.MESH)
            d.start()
            ag.append(d)
        ag[NH - 1].wait_recv()

        for d in rs:
            d.wait_send()
        for d in ag:
            d.wait_send()

        @functools.partial(pl.run_scoped,
                           exit_sem=pltpu.SemaphoreType.REGULAR)
        def _(exit_sem):
            for nbr in (left, right):
                pl.semaphore_signal(exit_sem, inc=1, device_id=(nbr,),
                                    device_id_type=pl.DeviceIdType.MESH)
            pl.semaphore_wait(exit_sem, 2)

    out = pl.pallas_call(
        body,
        out_shape=jax.ShapeDtypeStruct((TOK, D), jnp.float32),
        in_specs=[pl.BlockSpec(memory_space=pltpu.VMEM)] * 5,
        out_specs=pl.BlockSpec(memory_space=pltpu.VMEM),
        scratch_shapes=[
            pltpu.VMEM((TOK, D), jnp.float32),
            pltpu.VMEM((NH, CH, D), jnp.float32),
            pltpu.SemaphoreType.DMA((NH,)),
            pltpu.SemaphoreType.DMA((NH,)),
            pltpu.SemaphoreType.DMA((NH,)),
            pltpu.SemaphoreType.DMA((NH,)),
        ],
        compiler_params=pltpu.CompilerParams(collective_id=0),
    )(xf, Wq, Wk_s, Wv_s, Wo)
    return out.reshape(B, SQ, D)


# baseline (device time: 164919 ns/iter reference)
import functools

import jax
import jax.numpy as jnp
from jax import lax
from jax.experimental import pallas as pl
from jax.experimental.pallas import tpu as pltpu

N_DEV = 32
B, SQ, D, DH = 2, 256, 768, 64
HQ_LOCAL = 8
GQA = 4
KVCOLS = (HQ_LOCAL // GQA) * DH
TOK = B * SQ
CH = TOK // N_DEV
NH = N_DEV - 1


def kernel(x, Wq, Wo, Wk, Wv):
    me_out = lax.axis_index("i")
    xf = x.reshape(TOK, D)
    Wk_s = lax.dynamic_slice(Wk, (0, me_out * KVCOLS), (D, KVCOLS))
    Wv_s = lax.dynamic_slice(Wv, (0, me_out * KVCOLS), (D, KVCOLS))

    def body(x_ref, wq_ref, wk_ref, wv_ref, wo_ref, out_ref,
             part_ref, rbuf_ref, rs_send, rs_recv, ag_send, ag_recv):
        me = lax.axis_index("i")
        left = lax.rem(me + N_DEV - 1, N_DEV)
        right = lax.rem(me + 1, N_DEV)

        barrier = pltpu.get_barrier_semaphore()
        for nbr in (left, right):
            pl.semaphore_signal(barrier, inc=1, device_id=(nbr,),
                                device_id_type=pl.DeviceIdType.MESH)
        pl.semaphore_wait(barrier, 2)

        xv = x_ref[:, :]
        q = jnp.dot(xv, wq_ref[:, :], preferred_element_type=jnp.float32)
        kk = jnp.dot(xv, wk_ref[:, :], preferred_element_type=jnp.float32)
        vv = jnp.dot(xv, wv_ref[:, :], preferred_element_type=jnp.float32)
        o_rows = []
        for b in range(B):
            r0 = b * SQ
            heads = []
            for h in range(HQ_LOCAL):
                g = h // GQA
                qh = q[r0:r0 + SQ, h * DH:(h + 1) * DH]
                kh = kk[r0:r0 + SQ, g * DH:(g + 1) * DH]
                vh = vv[r0:r0 + SQ, g * DH:(g + 1) * DH]
                s = lax.dot_general(qh, kh, (((1,), (1,)), ((), ())),
                                    preferred_element_type=jnp.float32)
                s = s * 0.125
                m = jnp.max(s, axis=1, keepdims=True)
                p = jnp.exp(s - m)
                l = jnp.sum(p, axis=1, keepdims=True)
                o = jnp.dot(p, vh, preferred_element_type=jnp.float32) / l
                heads.append(o)
            o_rows.append(jnp.concatenate(heads, axis=1))
        o_full = jnp.concatenate(o_rows, axis=0)
        part = jnp.dot(o_full, wo_ref[:, :],
                       preferred_element_type=jnp.float32)
        part_ref[:, :, :] = part.reshape(N_DEV, CH, D)

        rs = []
        for t in range(NH):
            c = lax.rem(me - t + N_DEV, N_DEV)
            if t == 0:
                src = part_ref.at[c]
            else:
                rs[t - 1].wait_recv()
                rbuf_ref[t - 1, :, :] = rbuf_ref[t - 1, :, :] + part_ref[c]
                src = rbuf_ref.at[t - 1]
            d = pltpu.make_async_remote_copy(
                src_ref=src, dst_ref=rbuf_ref.at[t],
                send_sem=rs_send.at[t], recv_sem=rs_recv.at[t],
                device_id=(right,), device_id_type=pl.DeviceIdType.MESH)
            d.start()
            rs.append(d)
        rs[NH - 1].wait_recv()
        c_own = lax.rem(me + 1, N_DEV)
        out_ref[c_own, :, :] = rbuf_ref[NH - 1, :, :] + part_ref[c_own]

        ag = []
        for t in range(NH):
            c_src = lax.rem(me + 1 - t + N_DEV, N_DEV)
            if t >= 1:
                ag[t - 1].wait_recv()
            d = pltpu.make_async_remote_copy(
                src_ref=out_ref.at[c_src], dst_ref=out_ref.at[c_src],
                send_sem=ag_send.at[t], recv_sem=ag_recv.at[t],
                device_id=(right,), device_id_type=pl.DeviceIdType.MESH)
            d.start()
            ag.append(d)
        ag[NH - 1].wait_recv()

        for d in rs:
            d.wait_send()
        for d in ag:
            d.wait_send()

        @functools.partial(pl.run_scoped,
                           exit_sem=pltpu.SemaphoreType.REGULAR)
        def _(exit_sem):
            for nbr in (left, right):
                pl.semaphore_signal(exit_sem, inc=1, device_id=(nbr,),
                                    device_id_type=pl.DeviceIdType.MESH)
            pl.semaphore_wait(exit_sem, 2)

    out = pl.pallas_call(
        body,
        out_shape=jax.ShapeDtypeStruct((N_DEV, CH, D), jnp.float32),
        in_specs=[pl.BlockSpec(memory_space=pltpu.VMEM)] * 5,
        out_specs=pl.BlockSpec(memory_space=pltpu.VMEM),
        scratch_shapes=[
            pltpu.VMEM((N_DEV, CH, D), jnp.float32),
            pltpu.VMEM((NH, CH, D), jnp.float32),
            pltpu.SemaphoreType.DMA((NH,)),
            pltpu.SemaphoreType.DMA((NH,)),
            pltpu.SemaphoreType.DMA((NH,)),
            pltpu.SemaphoreType.DMA((NH,)),
        ],
        compiler_params=pltpu.CompilerParams(collective_id=0),
    )(xf, Wq, Wk_s, Wv_s, Wo)
    return out.reshape(B, SQ, D)


# device time: 88493 ns/iter; 1.8636x vs baseline; 1.8636x over previous
import functools

import jax
import jax.numpy as jnp
from jax import lax
from jax.experimental import pallas as pl
from jax.experimental.pallas import tpu as pltpu

N_DEV = 32
B, SQ, D, DH = 2, 256, 768, 64
HQ_LOCAL = 8
GQA = 4
KVCOLS = (HQ_LOCAL // GQA) * DH
TOK = B * SQ
CH = TOK // N_DEV
NST = 5
RS_OFF = (0, 16, 24, 28, 30)


def kernel(x, Wq, Wo, Wk, Wv):
    me_out = lax.axis_index("i")
    xf = x.reshape(TOK, D)
    Wk_s = lax.dynamic_slice(Wk, (0, me_out * KVCOLS), (D, KVCOLS))
    Wv_s = lax.dynamic_slice(Wv, (0, me_out * KVCOLS), (D, KVCOLS))

    def body(x_ref, wq_ref, wk_ref, wv_ref, wo_ref, out_ref,
             acc_ref, rbuf_ref, rs_send, rs_recv, ag_send, ag_recv):
        me = lax.axis_index("i")
        partners = [me ^ (1 << j) for j in range(NST)]

        barrier = pltpu.get_barrier_semaphore()
        for p in partners:
            pl.semaphore_signal(barrier, inc=1, device_id=(p,),
                                device_id_type=pl.DeviceIdType.MESH)
        pl.semaphore_wait(barrier, NST)

        xv = x_ref[:, :]
        q = jnp.dot(xv, wq_ref[:, :], preferred_element_type=jnp.float32)
        kk = jnp.dot(xv, wk_ref[:, :], preferred_element_type=jnp.float32)
        vv = jnp.dot(xv, wv_ref[:, :], preferred_element_type=jnp.float32)
        o_rows = []
        for b in range(B):
            r0 = b * SQ
            heads = []
            for h in range(HQ_LOCAL):
                g = h // GQA
                qh = q[r0:r0 + SQ, h * DH:(h + 1) * DH]
                kh = kk[r0:r0 + SQ, g * DH:(g + 1) * DH]
                vh = vv[r0:r0 + SQ, g * DH:(g + 1) * DH]
                s = lax.dot_general(qh, kh, (((1,), (1,)), ((), ())),
                                    preferred_element_type=jnp.float32)
                s = s * 0.125
                m = jnp.max(s, axis=1, keepdims=True)
                p_ = jnp.exp(s - m)
                l = jnp.sum(p_, axis=1, keepdims=True)
                o = jnp.dot(p_, vh, preferred_element_type=jnp.float32) / l
                heads.append(o)
            o_rows.append(jnp.concatenate(heads, axis=1))
        o_full = jnp.concatenate(o_rows, axis=0)
        part = jnp.dot(o_full, wo_ref[:, :],
                       preferred_element_type=jnp.float32)
        acc_ref[:, :, :] = part.reshape(N_DEV, CH, D)

        rs = []
        for s in range(NST):
            H = 16 >> s
            b0 = (me // (2 * H)) * (2 * H)
            bit = lax.rem(me // H, 2)
            keep = b0 + bit * H
            give = b0 + (1 - bit) * H
            d = pltpu.make_async_remote_copy(
                src_ref=acc_ref.at[pl.ds(give, H)],
                dst_ref=rbuf_ref.at[pl.ds(RS_OFF[s], H)],
                send_sem=rs_send.at[s], recv_sem=rs_recv.at[s],
                device_id=(me ^ H,), device_id_type=pl.DeviceIdType.MESH)
            d.start()
            d.wait_recv()
            acc_ref[pl.ds(keep, H), :, :] = (
                acc_ref[pl.ds(keep, H), :, :]
                + rbuf_ref[RS_OFF[s]:RS_OFF[s] + H, :, :])
            rs.append(d)
        out_ref[me, :, :] = acc_ref[me, :, :]

        ag = []
        for k in range(NST):
            L = 1 << k
            own = (me // L) * L
            d = pltpu.make_async_remote_copy(
                src_ref=out_ref.at[pl.ds(own, L)],
                dst_ref=out_ref.at[pl.ds(own, L)],
                send_sem=ag_send.at[k], recv_sem=ag_recv.at[k],
                device_id=(me ^ L,), device_id_type=pl.DeviceIdType.MESH)
            d.start()
            d.wait_recv()
            ag.append(d)

        for d in rs:
            d.wait_send()
        for d in ag:
            d.wait_send()

        @functools.partial(pl.run_scoped,
                           exit_sem=pltpu.SemaphoreType.REGULAR)
        def _(exit_sem):
            for p in partners:
                pl.semaphore_signal(exit_sem, inc=1, device_id=(p,),
                                    device_id_type=pl.DeviceIdType.MESH)
            pl.semaphore_wait(exit_sem, NST)

    out = pl.pallas_call(
        body,
        out_shape=jax.ShapeDtypeStruct((N_DEV, CH, D), jnp.float32),
        in_specs=[pl.BlockSpec(memory_space=pltpu.VMEM)] * 5,
        out_specs=pl.BlockSpec(memory_space=pltpu.VMEM),
        scratch_shapes=[
            pltpu.VMEM((N_DEV, CH, D), jnp.float32),
            pltpu.VMEM((31, CH, D), jnp.float32),
            pltpu.SemaphoreType.DMA((NST,)),
            pltpu.SemaphoreType.DMA((NST,)),
            pltpu.SemaphoreType.DMA((NST,)),
            pltpu.SemaphoreType.DMA((NST,)),
        ],
        compiler_params=pltpu.CompilerParams(collective_id=0),
    )(xf, Wq, Wk_s, Wv_s, Wo)
    return out.reshape(B, SQ, D)


# device time: 56889 ns/iter; 2.8990x vs baseline; 1.5555x over previous
import functools

import jax
import jax.numpy as jnp
from jax import lax
from jax.experimental import pallas as pl
from jax.experimental.pallas import tpu as pltpu

N_DEV = 32
B, SQ, D, DH = 2, 256, 768, 64
HQ_LOCAL = 8
GQA = 4
KVCOLS = (HQ_LOCAL // GQA) * DH
TOK = B * SQ
CH = TOK // N_DEV
NST = 5
RS_OFF = (0, 16, 24, 28, 30)


def kernel(x, Wq, Wo, Wk, Wv):
    me_out = lax.axis_index("i")
    xf = x.reshape(TOK, D)
    Wk_s = lax.dynamic_slice(Wk, (0, me_out * KVCOLS), (D, KVCOLS))
    Wv_s = lax.dynamic_slice(Wv, (0, me_out * KVCOLS), (D, KVCOLS))

    def body(x_ref, wq_ref, wk_ref, wv_ref, wo_ref, out_ref,
             acc_ref, sbuf_ref, rbuf_ref, agbuf_ref,
             rs_send, rs_recv, ag_send, ag_recv):
        me = lax.axis_index("i")
        partners = [me ^ (1 << j) for j in range(NST)]

        barrier = pltpu.get_barrier_semaphore()
        for p in partners:
            pl.semaphore_signal(barrier, inc=1, device_id=(p,),
                                device_id_type=pl.DeviceIdType.MESH)
        pl.semaphore_wait(barrier, NST)

        def compute_batch(cb):
            rb = cb * CH
            xb = x_ref[pl.ds(rb, SQ), :]
            qb = jnp.dot(xb, wq_ref[:, :], preferred_element_type=jnp.float32)
            kb = jnp.dot(xb, wk_ref[:, :], preferred_element_type=jnp.float32)
            vb = jnp.dot(xb, wv_ref[:, :], preferred_element_type=jnp.float32)
            heads = []
            for h in range(HQ_LOCAL):
                g = h // GQA
                qh = qb[:, h * DH:(h + 1) * DH]
                kh = kb[:, g * DH:(g + 1) * DH]
                vh = vb[:, g * DH:(g + 1) * DH]
                s = lax.dot_general(qh, kh, (((1,), (1,)), ((), ())),
                                    preferred_element_type=jnp.float32)
                s = s * 0.125
                m = jnp.max(s, axis=1, keepdims=True)
                p_ = jnp.exp(s - m)
                l = jnp.sum(p_, axis=1, keepdims=True)
                heads.append(
                    jnp.dot(p_, vh, preferred_element_type=jnp.float32) / l)
            ob = jnp.concatenate(heads, axis=1)
            return jnp.dot(ob, wo_ref[:, :],
                           preferred_element_type=jnp.float32)

        bit0 = lax.rem(me // 16, 2)
        cb_keep = bit0 * 16
        cb_give = (1 - bit0) * 16
        part_g = compute_batch(cb_give)
        acc_ref[pl.ds(cb_give, 16), :, :] = part_g.reshape(16, CH, D)
        sbuf_ref[pl.ds(RS_OFF[0], 16), :, :] = (
            part_g.astype(jnp.bfloat16).reshape(16, CH, D))
        rs = []
        d0 = pltpu.make_async_remote_copy(
            src_ref=sbuf_ref.at[pl.ds(RS_OFF[0], 16)],
            dst_ref=rbuf_ref.at[pl.ds(RS_OFF[0], 16)],
            send_sem=rs_send.at[0], recv_sem=rs_recv.at[0],
            device_id=(me ^ 16,), device_id_type=pl.DeviceIdType.MESH)
        d0.start()
        rs.append(d0)
        part_k = compute_batch(cb_keep)
        acc_ref[pl.ds(cb_keep, 16), :, :] = part_k.reshape(16, CH, D)
        d0.wait_recv()
        acc_ref[pl.ds(cb_keep, 16), :, :] = (
            acc_ref[pl.ds(cb_keep, 16), :, :]
            + rbuf_ref[RS_OFF[0]:RS_OFF[0] + 16, :, :].astype(jnp.float32))

        for s in range(1, NST):
            H = 16 >> s
            b0 = (me // (2 * H)) * (2 * H)
            bit = lax.rem(me // H, 2)
            keep = b0 + bit * H
            give = b0 + (1 - bit) * H
            sbuf_ref[pl.ds(RS_OFF[s], H), :, :] = (
                acc_ref[pl.ds(give, H), :, :].astype(jnp.bfloat16))
            d = pltpu.make_async_remote_copy(
                src_ref=sbuf_ref.at[pl.ds(RS_OFF[s], H)],
                dst_ref=rbuf_ref.at[pl.ds(RS_OFF[s], H)],
                send_sem=rs_send.at[s], recv_sem=rs_recv.at[s],
                device_id=(me ^ H,), device_id_type=pl.DeviceIdType.MESH)
            d.start()
            d.wait_recv()
            acc_ref[pl.ds(keep, H), :, :] = (
                acc_ref[pl.ds(keep, H), :, :]
                + rbuf_ref[RS_OFF[s]:RS_OFF[s] + H, :, :].astype(jnp.float32))
            rs.append(d)

        agbuf_ref[me, :, :] = acc_ref[me, :, :].astype(jnp.bfloat16)
        ag = []
        for k in range(NST):
            L = 1 << k
            own = (me // L) * L
            d = pltpu.make_async_remote_copy(
                src_ref=agbuf_ref.at[pl.ds(own, L)],
                dst_ref=agbuf_ref.at[pl.ds(own, L)],
                send_sem=ag_send.at[k], recv_sem=ag_recv.at[k],
                device_id=(me ^ L,), device_id_type=pl.DeviceIdType.MESH)
            d.start()
            d.wait_recv()
            ag.append(d)
        out_ref[:, :, :] = agbuf_ref[:, :, :].astype(jnp.float32)

        for d in rs:
            d.wait_send()
        for d in ag:
            d.wait_send()

        @functools.partial(pl.run_scoped,
                           exit_sem=pltpu.SemaphoreType.REGULAR)
        def _(exit_sem):
            for p in partners:
                pl.semaphore_signal(exit_sem, inc=1, device_id=(p,),
                                    device_id_type=pl.DeviceIdType.MESH)
            pl.semaphore_wait(exit_sem, NST)

    out = pl.pallas_call(
        body,
        out_shape=jax.ShapeDtypeStruct((N_DEV, CH, D), jnp.float32),
        in_specs=[pl.BlockSpec(memory_space=pltpu.VMEM)] * 5,
        out_specs=pl.BlockSpec(memory_space=pltpu.VMEM),
        scratch_shapes=[
            pltpu.VMEM((N_DEV, CH, D), jnp.float32),
            pltpu.VMEM((31, CH, D), jnp.bfloat16),
            pltpu.VMEM((31, CH, D), jnp.bfloat16),
            pltpu.VMEM((N_DEV, CH, D), jnp.bfloat16),
            pltpu.SemaphoreType.DMA((NST,)),
            pltpu.SemaphoreType.DMA((NST,)),
            pltpu.SemaphoreType.DMA((NST,)),
            pltpu.SemaphoreType.DMA((NST,)),
        ],
        compiler_params=pltpu.CompilerParams(collective_id=0),
    )(xf, Wq, Wk_s, Wv_s, Wo)
    return out.reshape(B, SQ, D)
